# baseline (speedup 1.0000x reference)
import functools

import jax
import jax.numpy as jnp
from jax import lax
from jax.experimental import pallas as pl
from jax.experimental.pallas import tpu as pltpu
from jax.experimental.pallas import tpu_sc as plsc

_NC = 2
_NS = 16
_NW = _NC * _NS
_B = 80
_RB = 256


def _deg_fn(Np, NB):
  rows_pt = Np // _NS
  NBH = NB // 2
  mesh = plsc.VectorSubcoreMesh(core_axis_name="c", subcore_axis_name="s")

  @functools.partial(
      pl.kernel,
      out_type=jax.ShapeDtypeStruct((_NC * Np,), jnp.float32),
      mesh=mesh,
      scratch_types=[
          pltpu.VMEM((2, NBH, _B), jnp.int32),
          pltpu.VMEM((_B,), jnp.float32),
          pltpu.VMEM((rows_pt,), jnp.float32),
          pltpu.VMEM_SHARED((Np,), jnp.float32),
      ],
  )
  def deg_k(dst_hbm, out_hbm, dst_v, ones_v, zeros_v, acc):
    c = lax.axis_index("c")
    s = lax.axis_index("s")
    wid = c * _NS + s
    pltpu.sync_copy(dst_hbm.at[wid], dst_v)

    def fill_ones(i, _):
      ones_v[pl.ds(i * 16, 16)] = jnp.full((16,), 1.0, jnp.float32)
      return 0

    lax.fori_loop(0, _B // 16, fill_ones, 0)

    def fill_zeros(i, _):
      zeros_v[pl.ds(i * 16, 16)] = jnp.zeros((16,), jnp.float32)
      return 0

    lax.fori_loop(0, rows_pt // 16, fill_zeros, 0)

    row0 = s * rows_pt
    pltpu.sync_copy(zeros_v, acc.at[pl.ds(row0, rows_pt)])
    plsc.subcore_barrier()

    for h in range(2):
      def body(i, _):
        pltpu.sync_copy(ones_v, acc.at[dst_v.at[h, i]], add=True)
        return 0

      lax.fori_loop(0, NBH, body, 0)
    plsc.subcore_barrier()
    pltpu.sync_copy(acc.at[pl.ds(row0, rows_pt)],
                    out_hbm.at[pl.ds(c * Np + row0, rows_pt)])

  return deg_k


def _agg_fn(Np, D, NB):
  rows_pt = Np // _NS
  mesh = plsc.VectorSubcoreMesh(core_axis_name="c", subcore_axis_name="s")

  NBH = NB // 2

  @functools.partial(
      pl.kernel,
      out_type=jax.ShapeDtypeStruct((_NC * Np, D), jnp.float32),
      mesh=mesh,
      scratch_types=[
          pltpu.VMEM((NBH, _B), jnp.int32),
          pltpu.VMEM((NBH, _B), jnp.int32),
          pltpu.VMEM((_B, D), jnp.float32),
          pltpu.VMEM((_B, D), jnp.float32),
          pltpu.VMEM((_B, D), jnp.float32),
          pltpu.VMEM_SHARED((Np, D), jnp.float32),
          pltpu.SemaphoreType.DMA,
          pltpu.SemaphoreType.DMA,
          pltpu.SemaphoreType.DMA,
      ],
  )
  def agg_k(h_hbm, src_hbm, dst_hbm, out_hbm, idxs, idxd, buf0, buf1, buf2,
            acc, sem0, sem1, sem2):
    c = lax.axis_index("c")
    s = lax.axis_index("s")
    wid = c * _NS + s
    bufs = (buf0, buf1, buf2)
    sems = (sem0, sem1, sem2)

    nvec = D // 16

    def fill_zeros(r, _):
      for j in range(nvec):
        buf0[r, pl.ds(j * 16, 16)] = jnp.zeros((16,), jnp.float32)
      return 0

    lax.fori_loop(0, _B, fill_zeros, 0)

    row0 = s * rows_pt
    zfull = rows_pt // _B
    zrem = rows_pt % _B

    def zero_acc(i, _):
      pltpu.sync_copy(buf0, acc.at[pl.ds(row0 + i * _B, _B)])
      return 0

    lax.fori_loop(0, zfull, zero_acc, 0)
    if zrem:
      pltpu.sync_copy(buf0.at[pl.ds(0, zrem)],
                      acc.at[pl.ds(row0 + zfull * _B, zrem)])

    def body(k, _):
      for j in range(3):
        i = 3 * k + j
        pltpu.make_async_copy(h_hbm.at[pl.ds(0, _B)], bufs[j], sems[j]).wait()

        @pl.when(i + 2 < NBH)
        def _():
          pltpu.async_copy(h_hbm.at[idxs.at[i + 2]], bufs[(j + 2) % 3],
                           sems[(j + 2) % 3])

        pltpu.sync_copy(bufs[j], acc.at[idxd.at[i]], add=True)
      return 0

    for h in range(2):
      pltpu.sync_copy(src_hbm.at[wid, h], idxs)
      pltpu.sync_copy(dst_hbm.at[wid, h], idxd)
      pltpu.async_copy(h_hbm.at[idxs.at[0]], buf0, sem0)
      pltpu.async_copy(h_hbm.at[idxs.at[1]], buf1, sem1)
      if h == 0:
        plsc.subcore_barrier()
      lax.fori_loop(0, NBH // 3, body, 0)

    plsc.subcore_barrier()
    pltpu.sync_copy(acc.at[pl.ds(row0, rows_pt)],
                    out_hbm.at[pl.ds(c * Np + row0, rows_pt)])

  return agg_k


def _tc_mm(xp, W):
  Np, D = xp.shape
  g = Np // _RB

  def body(x_ref, w_ref, o_ref):
    o_ref[...] = jnp.dot(x_ref[...], w_ref[...],
                         preferred_element_type=jnp.float32)

  return pl.pallas_call(
      body,
      grid=(g,),
      in_specs=[pl.BlockSpec((_RB, D), lambda i: (i, 0)),
                pl.BlockSpec((D, D), lambda i: (0, 0))],
      out_specs=pl.BlockSpec((_RB, D), lambda i: (i, 0)),
      out_shape=jax.ShapeDtypeStruct((Np, D), jnp.float32),
  )(xp, W)


def _tc_scale(h, dinv):
  Np, D = h.shape
  g = Np // _RB

  def body(h_ref, d_ref, o_ref):
    o_ref[...] = h_ref[...] * d_ref[...]

  return pl.pallas_call(
      body,
      grid=(g,),
      in_specs=[pl.BlockSpec((_RB, D), lambda i: (i, 0)),
                pl.BlockSpec((_RB, 1), lambda i: (i, 0))],
      out_specs=pl.BlockSpec((_RB, D), lambda i: (i, 0)),
      out_shape=jax.ShapeDtypeStruct((Np, D), jnp.float32),
  )(h, dinv)


def _tc_mid(aggf, h1p, dinv, b1r, gsc, betar, W2):
  Np, D = h1p.shape
  g = Np // _RB
  off = g

  def body(a0, a1, hp, d, b1_, g_, be_, w_, o):
    pre = (a0[...] + a1[...] + hp[...]) * d[...] + b1_[...]
    z = jnp.maximum(pre * g_[...] + be_[...], 0.0)
    o[...] = jnp.dot(z, w_[...], preferred_element_type=jnp.float32) * d[...]

  return pl.pallas_call(
      body,
      grid=(g,),
      in_specs=[pl.BlockSpec((_RB, D), lambda i: (i, 0)),
                pl.BlockSpec((_RB, D), lambda i: (i + off, 0)),
                pl.BlockSpec((_RB, D), lambda i: (i, 0)),
                pl.BlockSpec((_RB, 1), lambda i: (i, 0)),
                pl.BlockSpec((1, D), lambda i: (0, 0)),
                pl.BlockSpec((1, D), lambda i: (0, 0)),
                pl.BlockSpec((1, D), lambda i: (0, 0)),
                pl.BlockSpec((D, D), lambda i: (0, 0))],
      out_specs=pl.BlockSpec((_RB, D), lambda i: (i, 0)),
      out_shape=jax.ShapeDtypeStruct((Np, D), jnp.float32),
  )(aggf, aggf, h1p, dinv, b1r, gsc, betar, W2)


def _tc_out(aggf, h2p, dinv, b2r, N):
  Np, D = h2p.shape
  g = Np // _RB
  off = g

  def body(a0, a1, hp, d, b2_, o):
    pre = (a0[...] + a1[...] + hp[...]) * d[...] + b2_[...]
    m = jnp.max(pre, axis=-1, keepdims=True)
    e = jnp.exp(pre - m)
    lse = jnp.log(jnp.sum(e, axis=-1, keepdims=True)) + m
    o[...] = pre - lse

  return pl.pallas_call(
      body,
      grid=(g,),
      in_specs=[pl.BlockSpec((_RB, D), lambda i: (i, 0)),
                pl.BlockSpec((_RB, D), lambda i: (i + off, 0)),
                pl.BlockSpec((_RB, D), lambda i: (i, 0)),
                pl.BlockSpec((_RB, 1), lambda i: (i, 0)),
                pl.BlockSpec((1, D), lambda i: (0, 0))],
      out_specs=pl.BlockSpec((_RB, D), lambda i: (i, 0)),
      out_shape=jax.ShapeDtypeStruct((N, D), jnp.float32),
  )(aggf, aggf, h2p, dinv, b2r)


def kernel(x, edge_index, W1, b1, gamma, beta, W2, b2):
  N, D = x.shape
  E = edge_index.shape[1]
  Np = _RB * ((N + _RB - 1) // _RB)
  if Np == N:
    Np += _RB
  NB = (E + _NW * _B - 1) // (_NW * _B)
  NB = 6 * ((NB + 5) // 6)
  NBH = NB // 2
  Ep = _NW * _B * NB
  pad = Ep - E

  src = edge_index[0].astype(jnp.int32)
  dst = edge_index[1].astype(jnp.int32)
  ar = jnp.arange(pad, dtype=jnp.int32)
  src_w = jnp.concatenate([src, ar % N]).reshape(_NW, 2, NBH, _B)
  dst_w = jnp.concatenate([dst, N + ar % (Np - N)]).reshape(_NW, 2, NBH, _B)

  xp = jnp.pad(x, ((0, Np - N), (0, 0)))

  h1 = _tc_mm(xp, W1)
  degp = _deg_fn(Np, NB)(dst_w)
  deg = degp[:Np] + degp[Np:] + 1.0
  dinv = lax.rsqrt(deg)[:, None]

  b1r = b1.reshape(1, D)
  b2r = b2.reshape(1, D)
  gsc = (gamma * (1.0 / jnp.sqrt(1.0 + 1e-5))).reshape(1, D)
  betar = beta.reshape(1, D)

  agg = _agg_fn(Np, D, NB)

  h1p = _tc_scale(h1, dinv)
  a1 = agg(h1p, src_w, dst_w)
  h2p = _tc_mid(a1, h1p, dinv, b1r, gsc, betar, W2)
  a2 = agg(h2p, src_w, dst_w)
  return _tc_out(a2, h2p, dinv, b2r, N)


# speedup vs baseline: 1.0463x; 1.0463x over previous
import functools

import jax
import jax.numpy as jnp
from jax import lax
from jax.experimental import pallas as pl
from jax.experimental.pallas import tpu as pltpu
from jax.experimental.pallas import tpu_sc as plsc

_NC = 2
_NS = 16
_NW = _NC * _NS
_B = 80
_RB = 256


def _deg_fn(Np, NB):
  rows_pt = Np // _NS
  NBH = NB // 2
  mesh = plsc.VectorSubcoreMesh(core_axis_name="c", subcore_axis_name="s")

  @functools.partial(
      pl.kernel,
      out_type=jax.ShapeDtypeStruct((_NC * Np,), jnp.float32),
      mesh=mesh,
      scratch_types=[
          pltpu.VMEM((2, NBH, _B), jnp.int32),
          pltpu.VMEM((_B,), jnp.float32),
          pltpu.VMEM((rows_pt,), jnp.float32),
          pltpu.VMEM_SHARED((Np,), jnp.float32),
      ],
  )
  def deg_k(dst_hbm, out_hbm, dst_v, ones_v, zeros_v, acc):
    c = lax.axis_index("c")
    s = lax.axis_index("s")
    wid = c * _NS + s
    pltpu.sync_copy(dst_hbm.at[wid], dst_v)

    def fill_ones(i, _):
      ones_v[pl.ds(i * 16, 16)] = jnp.full((16,), 1.0, jnp.float32)
      return 0

    lax.fori_loop(0, _B // 16, fill_ones, 0)

    def fill_zeros(i, _):
      zeros_v[pl.ds(i * 16, 16)] = jnp.zeros((16,), jnp.float32)
      return 0

    lax.fori_loop(0, rows_pt // 16, fill_zeros, 0)

    row0 = s * rows_pt
    pltpu.sync_copy(zeros_v, acc.at[pl.ds(row0, rows_pt)])
    plsc.subcore_barrier()

    for h in range(2):
      def body(i, _):
        pltpu.sync_copy(ones_v, acc.at[dst_v.at[h, i]], add=True)
        return 0

      lax.fori_loop(0, NBH, body, 0)
    plsc.subcore_barrier()
    pltpu.sync_copy(acc.at[pl.ds(row0, rows_pt)],
                    out_hbm.at[pl.ds(c * Np + row0, rows_pt)])

  return deg_k


def _agg_fn(Np, D, NB):
  rows_pt = Np // _NS
  mesh = plsc.VectorSubcoreMesh(core_axis_name="c", subcore_axis_name="s")

  NBH = NB // 2

  @functools.partial(
      pl.kernel,
      out_type=jax.ShapeDtypeStruct((_NC * Np, D), jnp.float32),
      mesh=mesh,
      scratch_types=[
          pltpu.VMEM((NBH, _B), jnp.int32),
          pltpu.VMEM((NBH, _B), jnp.int32),
          pltpu.VMEM((_B, D), jnp.float32),
          pltpu.VMEM((_B, D), jnp.float32),
          pltpu.VMEM((_B, D), jnp.float32),
          pltpu.VMEM_SHARED((Np, D), jnp.float32),
          pltpu.SemaphoreType.DMA,
          pltpu.SemaphoreType.DMA,
          pltpu.SemaphoreType.DMA,
      ],
  )
  def agg_k(h_hbm, src_hbm, dst_hbm, out_hbm, idxs, idxd, buf0, buf1, buf2,
            acc, sem0, sem1, sem2):
    c = lax.axis_index("c")
    s = lax.axis_index("s")
    wid = c * _NS + s
    bufs = (buf0, buf1, buf2)
    sems = (sem0, sem1, sem2)

    nvec = D // 16

    def fill_zeros(r, _):
      for j in range(nvec):
        buf0[r, pl.ds(j * 16, 16)] = jnp.zeros((16,), jnp.float32)
      return 0

    lax.fori_loop(0, _B, fill_zeros, 0)

    row0 = s * rows_pt
    zfull = rows_pt // _B
    zrem = rows_pt % _B

    def zero_acc(i, _):
      pltpu.sync_copy(buf0, acc.at[pl.ds(row0 + i * _B, _B)])
      return 0

    lax.fori_loop(0, zfull, zero_acc, 0)
    if zrem:
      pltpu.sync_copy(buf0.at[pl.ds(0, zrem)],
                      acc.at[pl.ds(row0 + zfull * _B, zrem)])

    def body(k, _):
      for j in range(3):
        i = 3 * k + j
        pltpu.make_async_copy(h_hbm.at[pl.ds(0, _B)], bufs[j], sems[j]).wait()

        @pl.when(i + 2 < NBH)
        def _():
          pltpu.async_copy(h_hbm.at[idxs.at[i + 2]], bufs[(j + 2) % 3],
                           sems[(j + 2) % 3])

        pltpu.sync_copy(bufs[j], acc.at[idxd.at[i]], add=True)
      return 0

    for h in range(2):
      pltpu.sync_copy(src_hbm.at[wid, h], idxs)
      pltpu.sync_copy(dst_hbm.at[wid, h], idxd)
      pltpu.async_copy(h_hbm.at[idxs.at[0]], buf0, sem0)
      pltpu.async_copy(h_hbm.at[idxs.at[1]], buf1, sem1)
      if h == 0:
        plsc.subcore_barrier()
      lax.fori_loop(0, NBH // 3, body, 0)

    plsc.subcore_barrier()
    pltpu.sync_copy(acc.at[pl.ds(row0, rows_pt)],
                    out_hbm.at[pl.ds(c * Np + row0, rows_pt)])

  return agg_k


def _tc_mm_scale(xp, W, dinv):
  Np, D = xp.shape
  g = Np // _RB

  def body(x_ref, w_ref, d_ref, o_ref):
    o_ref[...] = jnp.dot(x_ref[...], w_ref[...],
                         preferred_element_type=jnp.float32) * d_ref[...]

  return pl.pallas_call(
      body,
      grid=(g,),
      in_specs=[pl.BlockSpec((_RB, D), lambda i: (i, 0)),
                pl.BlockSpec((D, D), lambda i: (0, 0)),
                pl.BlockSpec((_RB, 1), lambda i: (i, 0))],
      out_specs=pl.BlockSpec((_RB, D), lambda i: (i, 0)),
      out_shape=jax.ShapeDtypeStruct((Np, D), jnp.float32),
  )(xp, W, dinv)


def _tc_mid(aggf, h1p, dinv, b1r, gsc, betar, W2):
  Np, D = h1p.shape
  g = Np // _RB
  off = g

  def body(a0, a1, hp, d, b1_, g_, be_, w_, o):
    pre = (a0[...] + a1[...] + hp[...]) * d[...] + b1_[...]
    z = jnp.maximum(pre * g_[...] + be_[...], 0.0)
    o[...] = jnp.dot(z, w_[...], preferred_element_type=jnp.float32) * d[...]

  return pl.pallas_call(
      body,
      grid=(g,),
      in_specs=[pl.BlockSpec((_RB, D), lambda i: (i, 0)),
                pl.BlockSpec((_RB, D), lambda i: (i + off, 0)),
                pl.BlockSpec((_RB, D), lambda i: (i, 0)),
                pl.BlockSpec((_RB, 1), lambda i: (i, 0)),
                pl.BlockSpec((1, D), lambda i: (0, 0)),
                pl.BlockSpec((1, D), lambda i: (0, 0)),
                pl.BlockSpec((1, D), lambda i: (0, 0)),
                pl.BlockSpec((D, D), lambda i: (0, 0))],
      out_specs=pl.BlockSpec((_RB, D), lambda i: (i, 0)),
      out_shape=jax.ShapeDtypeStruct((Np, D), jnp.float32),
  )(aggf, aggf, h1p, dinv, b1r, gsc, betar, W2)


def _tc_out(aggf, h2p, dinv, b2r, N):
  Np, D = h2p.shape
  g = Np // _RB
  off = g

  def body(a0, a1, hp, d, b2_, o):
    pre = (a0[...] + a1[...] + hp[...]) * d[...] + b2_[...]
    m = jnp.max(pre, axis=-1, keepdims=True)
    e = jnp.exp(pre - m)
    lse = jnp.log(jnp.sum(e, axis=-1, keepdims=True)) + m
    o[...] = pre - lse

  return pl.pallas_call(
      body,
      grid=(g,),
      in_specs=[pl.BlockSpec((_RB, D), lambda i: (i, 0)),
                pl.BlockSpec((_RB, D), lambda i: (i + off, 0)),
                pl.BlockSpec((_RB, D), lambda i: (i, 0)),
                pl.BlockSpec((_RB, 1), lambda i: (i, 0)),
                pl.BlockSpec((1, D), lambda i: (0, 0))],
      out_specs=pl.BlockSpec((_RB, D), lambda i: (i, 0)),
      out_shape=jax.ShapeDtypeStruct((N, D), jnp.float32),
  )(aggf, aggf, h2p, dinv, b2r)


def kernel(x, edge_index, W1, b1, gamma, beta, W2, b2):
  N, D = x.shape
  E = edge_index.shape[1]
  Np = _RB * ((N + _RB - 1) // _RB)
  if Np == N:
    Np += _RB
  NB = (E + _NW * _B - 1) // (_NW * _B)
  NB = 6 * ((NB + 5) // 6)
  NBH = NB // 2
  Ep = _NW * _B * NB
  pad = Ep - E

  src = edge_index[0].astype(jnp.int32)
  dst = edge_index[1].astype(jnp.int32)
  ar = jnp.arange(pad, dtype=jnp.int32)
  src_w = jnp.concatenate([src, ar % N]).reshape(_NW, 2, NBH, _B)
  dst_w = jnp.concatenate([dst, N + ar % (Np - N)]).reshape(_NW, 2, NBH, _B)

  xp = jnp.pad(x, ((0, Np - N), (0, 0)))

  degp = _deg_fn(Np, NB)(dst_w)
  deg = degp[:Np] + degp[Np:] + 1.0
  dinv = lax.rsqrt(deg)[:, None]

  b1r = b1.reshape(1, D)
  b2r = b2.reshape(1, D)
  gsc = (gamma * (1.0 / jnp.sqrt(1.0 + 1e-5))).reshape(1, D)
  betar = beta.reshape(1, D)

  agg = _agg_fn(Np, D, NB)

  h1p = _tc_mm_scale(xp, W1, dinv)
  a1 = agg(h1p, src_w, dst_w)
  h2p = _tc_mid(a1, h1p, dinv, b1r, gsc, betar, W2)
  a2 = agg(h2p, src_w, dst_w)
  return _tc_out(a2, h2p, dinv, b2r, N)
